# baseline (speedup 1.0000x reference)
import functools
import jax
import jax.numpy as jnp
from jax import lax
from jax.experimental import pallas as pl
from jax.experimental.pallas import tpu as pltpu
from jax.experimental.pallas import tpu_sc as plsc


def _sc_copy_fn(seq_len, embed_dim, dtype):
    info = plsc.get_sparse_core_info()
    nc, ns = info.num_cores, info.num_subcores
    nw = nc * ns
    rows_per_w = seq_len // nw
    q = 4
    chunk = rows_per_w // q
    mesh = plsc.VectorSubcoreMesh(core_axis_name="c", subcore_axis_name="s")

    @functools.partial(
        pl.kernel,
        mesh=mesh,
        out_type=jax.ShapeDtypeStruct((seq_len, embed_dim), dtype),
        scratch_types=[pltpu.SemaphoreType.DMA] * q,
    )
    def sc_copy(table_hbm, out_hbm, *sems):
        wid = lax.axis_index("s") * nc + lax.axis_index("c")
        base = wid * rows_per_w
        copies = []
        for g in range(q):
            copies.append(pltpu.async_copy(
                table_hbm.at[pl.ds(base + g * chunk, chunk)],
                out_hbm.at[pl.ds(base + g * chunk, chunk)],
                sems[g]))
        for c in copies:
            c.wait()

    return sc_copy


def kernel(token_ids, pos_table):
    seq_len = token_ids.shape[-1]
    embed_dim = pos_table.shape[1]
    fn = _sc_copy_fn(seq_len, embed_dim, pos_table.dtype)
    return fn(pos_table)


# speedup vs baseline: 32.0091x; 32.0091x over previous
import functools
import jax
import jax.numpy as jnp
from jax import lax
from jax.experimental import pallas as pl
from jax.experimental.pallas import tpu as pltpu
from jax.experimental.pallas import tpu_sc as plsc


def _sc_copy_fn(seq_len, embed_dim, dtype):
    info = plsc.get_sparse_core_info()
    nc, ns = info.num_cores, info.num_subcores
    nw = nc * ns
    rows_per_w = seq_len // nw
    chunk = 16
    nbuf = 3
    n_chunks = rows_per_w // chunk
    mesh = plsc.VectorSubcoreMesh(core_axis_name="c", subcore_axis_name="s")

    @functools.partial(
        pl.kernel,
        mesh=mesh,
        out_type=jax.ShapeDtypeStruct((seq_len, embed_dim), dtype),
        scratch_types=(
            [pltpu.VMEM_SHARED((ns, nbuf, chunk, embed_dim), dtype)]
            + [pltpu.SemaphoreType.DMA] * 6
        ),
    )
    def sc_copy(table_hbm, out_hbm, shared, *sems):
        rd_sems, wr_sems = sems[:3], sems[3:]
        sid = lax.axis_index("s")
        wid = sid * nc + lax.axis_index("c")
        base = wid * rows_per_w

        def read(g):
            s = g % nbuf
            return pltpu.async_copy(
                table_hbm.at[pl.ds(base + g * chunk, chunk)],
                shared.at[sid, s], rd_sems[s])

        def write(g):
            s = g % nbuf
            return pltpu.async_copy(
                shared.at[sid, s],
                out_hbm.at[pl.ds(base + g * chunk, chunk)], wr_sems[s])

        reads, writes = {}, {}
        pending_writes = set()
        ahead = nbuf - 2
        for g in range(min(ahead, n_chunks)):
            reads[g] = read(g)
        for g in range(n_chunks):
            nx = g + ahead
            if nx < n_chunks:
                prev = nx - nbuf
                if prev >= 0:
                    writes[prev].wait()
                    pending_writes.discard(prev)
                reads[nx] = read(nx)
            reads[g].wait()
            writes[g] = write(g)
            pending_writes.add(g)
        for g in sorted(pending_writes):
            writes[g].wait()

    return sc_copy


def kernel(token_ids, pos_table):
    seq_len = token_ids.shape[-1]
    embed_dim = pos_table.shape[1]
    fn = _sc_copy_fn(seq_len, embed_dim, pos_table.dtype)
    return fn(pos_table)
